# baseline (speedup 1.0000x reference)
import jax
import jax.numpy as jnp
from jax import lax
from jax.experimental import pallas as pl
from jax.experimental.pallas import tpu as pltpu

_NSHELF = 64
_SPATIAL = 128
_SLOPE = 0.01
_T = 2048
_R = _T // 128
_C = 128


def _leaky(v):
    return jnp.maximum(v, _SLOPE * v)


def _body(st_ref, st2_ref, x_ref, vw1_ref, vb1_ref, vw2_ref, vb2_ref,
          sw1_ref, sb1_ref, sw2_ref, sb2_ref, out_hbm, obuf, sem):
    f32 = jnp.float32
    i = pl.program_id(0)
    nblk = pl.num_programs(0)
    slot = lax.rem(i, 2)

    mf = (st2_ref[...] == _NSHELF).astype(f32)
    iu0 = lax.broadcasted_iota(jnp.int32, (128, 128), 0)
    iu1 = lax.broadcasted_iota(jnp.int32, (128, 128), 1)
    upper = jnp.where(iu0 <= iu1, 1.0, 0.0).astype(f32)
    rowcum = jnp.dot(mf, upper, preferred_element_type=f32)
    rowtot = rowcum[:, 127:128]
    is0 = lax.broadcasted_iota(jnp.int32, (_R, _R), 0)
    is1 = lax.broadcasted_iota(jnp.int32, (_R, _R), 1)
    strict = jnp.where(is1 < is0, 1.0, 0.0).astype(f32)
    prefix = jnp.dot(strict, rowtot, preferred_element_type=f32)
    rank = rowcum + prefix - mf
    rank = jnp.where(mf > 0, rank, -1.0)

    iota_c = lax.broadcasted_iota(jnp.int32, (_C, 128), 0).astype(f32)
    pieces = []
    for r in range(_R):
        rr = jnp.broadcast_to(rank[r:r + 1, :], (_C, 128))
        pieces.append(jnp.where(rr == iota_c, 1.0, 0.0))
    onehot = jnp.concatenate(pieces, axis=1).astype(f32)

    x = x_ref[...]
    gx = jnp.dot(onehot, x, preferred_element_type=f32)
    hv = _leaky(jnp.dot(gx[:, :_SPATIAL], vw1_ref[...],
                        preferred_element_type=f32) + vb1_ref[...])
    zv = jnp.dot(hv, vw2_ref[...], preferred_element_type=f32) + vb2_ref[...]

    hs = _leaky(jnp.dot(x, sw1_ref[...], preferred_element_type=f32)
                + sb1_ref[...])
    zs = jnp.dot(hs, sw2_ref[...], preferred_element_type=f32) + sb2_ref[...]

    iota_cl = lax.broadcasted_iota(jnp.int32, (128, _C), 1).astype(f32)
    rank_col = jnp.transpose(rank)
    pieces_t = []
    for r in range(_R):
        rr = jnp.broadcast_to(rank_col[:, r:r + 1], (128, _C))
        pieces_t.append(jnp.where(rr == iota_cl, 1.0, 0.0))
    onehot_t = jnp.concatenate(pieces_t, axis=0).astype(f32)
    scat = jnp.dot(onehot_t, zv, preferred_element_type=f32)
    mask_col = st_ref[...] == _NSHELF
    res = _leaky(jnp.where(mask_col, scat, zs))

    @pl.when(i >= 2)
    def _drain_prev():
        pltpu.make_async_copy(
            obuf.at[slot], out_hbm.at[pl.ds((i - 2) * _T, _T)],
            sem.at[slot]).wait()

    obuf[slot] = res
    pltpu.make_async_copy(
        obuf.at[slot], out_hbm.at[pl.ds(i * _T, _T)], sem.at[slot]).start()

    @pl.when(i == nblk - 1)
    def _drain_tail():
        @pl.when(i >= 1)
        def _():
            pltpu.make_async_copy(
                obuf.at[1 - slot], out_hbm.at[pl.ds((i - 1) * _T, _T)],
                sem.at[1 - slot]).wait()
        pltpu.make_async_copy(
            obuf.at[slot], out_hbm.at[pl.ds(i * _T, _T)], sem.at[slot]).wait()


def kernel(state, x, vw1, vb1, vw2, vb2, sw1, sb1, sw2, sb2):
    B, Nv, F = x.shape
    n_tok = B * Nv
    st = state.reshape(n_tok, 1).astype(jnp.int32)
    st2 = state.reshape(n_tok // 128, 128).astype(jnp.int32)
    xf = x.reshape(n_tok, F)
    grid = (n_tok // _T,)
    full = lambda shape: pl.BlockSpec(shape, lambda i: (0, 0))
    hbm = pl.BlockSpec(memory_space=pltpu.MemorySpace.HBM)
    out = pl.pallas_call(
        _body,
        grid=grid,
        in_specs=[
            pl.BlockSpec((_T, 1), lambda i: (i, 0)),
            pl.BlockSpec((_R, 128), lambda i: (i, 0)),
            pl.BlockSpec((_T, F), lambda i: (i, 0)),
            full(vw1.shape), full((1, vb1.shape[0])),
            full(vw2.shape), full((1, vb2.shape[0])),
            full(sw1.shape), full((1, sb1.shape[0])),
            full(sw2.shape), full((1, sb2.shape[0])),
        ],
        out_specs=pl.BlockSpec(memory_space=pltpu.MemorySpace.HBM),
        out_shape=jax.ShapeDtypeStruct((n_tok, 512), jnp.float32),
        scratch_shapes=[
            pltpu.VMEM((2, _T, 512), jnp.float32),
            pltpu.SemaphoreType.DMA((2,)),
        ],
        compiler_params=pltpu.CompilerParams(
            dimension_semantics=("arbitrary",)),
    )(st, st2, xf, vw1, vb1.reshape(1, -1), vw2, vb2.reshape(1, -1),
      sw1, sb1.reshape(1, -1), sw2, sb2.reshape(1, -1))
    return out.reshape(B, Nv, 512)


# speedup vs baseline: 1.1395x; 1.1395x over previous
import jax
import jax.numpy as jnp
from jax import lax
from jax.experimental import pallas as pl
from jax.experimental.pallas import tpu as pltpu

_NSHELF = 64
_SPATIAL = 128
_SLOPE = 0.01
_T = 4096
_R = _T // 128
_C = 128


def _leaky(v):
    return jnp.maximum(v, _SLOPE * v)


def _body(st_ref, st2_ref, x_ref, vw1_ref, vb1_ref, vw2_ref, vb2_ref,
          sw1_ref, sb1_ref, sw2_ref, sb2_ref, out_ref):
    f32 = jnp.float32

    mf = (st2_ref[...] == _NSHELF).astype(f32)
    iu0 = lax.broadcasted_iota(jnp.int32, (128, 128), 0)
    iu1 = lax.broadcasted_iota(jnp.int32, (128, 128), 1)
    upper = jnp.where(iu0 <= iu1, 1.0, 0.0).astype(f32)
    rowcum = jnp.dot(mf, upper, preferred_element_type=f32)
    rowtot = rowcum[:, 127:128]
    is0 = lax.broadcasted_iota(jnp.int32, (_R, _R), 0)
    is1 = lax.broadcasted_iota(jnp.int32, (_R, _R), 1)
    strict = jnp.where(is1 < is0, 1.0, 0.0).astype(f32)
    prefix = jnp.dot(strict, rowtot, preferred_element_type=f32)
    rank = rowcum + prefix - mf
    rank = jnp.where(mf > 0, rank, -1.0)

    bf = jnp.bfloat16

    iota_c = lax.broadcasted_iota(jnp.int32, (_C, 128), 0).astype(f32)
    pieces = []
    for r in range(_R):
        rr = jnp.broadcast_to(rank[r:r + 1, :], (_C, 128))
        pieces.append(jnp.where(rr == iota_c, 1.0, 0.0).astype(bf))
    onehot = jnp.concatenate(pieces, axis=1)

    xb = x_ref[...].astype(bf)
    gx = jnp.dot(onehot, xb, preferred_element_type=f32)
    hv = _leaky(jnp.dot(gx[:, :_SPATIAL].astype(bf), vw1_ref[...],
                        preferred_element_type=f32) + vb1_ref[...])
    zv = jnp.dot(hv.astype(bf), vw2_ref[...],
                 preferred_element_type=f32) + vb2_ref[...]

    hs = _leaky(jnp.dot(xb, sw1_ref[...], preferred_element_type=f32)
                + sb1_ref[...])
    zs = jnp.dot(hs.astype(bf), sw2_ref[...],
                 preferred_element_type=f32) + sb2_ref[...]

    iota_cl = lax.broadcasted_iota(jnp.int32, (128, _C), 1).astype(f32)
    rank_col = jnp.transpose(rank)
    pieces_t = []
    for r in range(_R):
        rr = jnp.broadcast_to(rank_col[:, r:r + 1], (128, _C))
        pieces_t.append(jnp.where(rr == iota_cl, 1.0, 0.0).astype(bf))
    onehot_t = jnp.concatenate(pieces_t, axis=0)
    scat = jnp.dot(onehot_t, zv.astype(bf),
                   preferred_element_type=f32)
    mask_col = st_ref[...] == _NSHELF
    out_ref[...] = _leaky(jnp.where(mask_col, scat, zs))


def kernel(state, x, vw1, vb1, vw2, vb2, sw1, sb1, sw2, sb2):
    B, Nv, F = x.shape
    n_tok = B * Nv
    st = state.reshape(n_tok, 1).astype(jnp.int32)
    st2 = state.reshape(n_tok // 128, 128).astype(jnp.int32)
    xf = x.reshape(n_tok, F)
    grid = (n_tok // _T,)
    full = lambda shape: pl.BlockSpec(shape, lambda i: (0, 0))
    hbm = pl.BlockSpec(memory_space=pltpu.MemorySpace.HBM)
    out = pl.pallas_call(
        _body,
        grid=grid,
        in_specs=[
            pl.BlockSpec((_T, 1), lambda i: (i, 0)),
            pl.BlockSpec((_R, 128), lambda i: (i, 0)),
            pl.BlockSpec((_T, F), lambda i: (i, 0)),
            full(vw1.shape), full((1, vb1.shape[0])),
            full(vw2.shape), full((1, vb2.shape[0])),
            full(sw1.shape), full((1, sb1.shape[0])),
            full(sw2.shape), full((1, sb2.shape[0])),
        ],
        out_specs=pl.BlockSpec((_T, 512), lambda i: (i, 0)),
        out_shape=jax.ShapeDtypeStruct((n_tok, 512), jnp.float32),
        compiler_params=pltpu.CompilerParams(
            dimension_semantics=("parallel",)),
    )(st, st2, xf, vw1.astype(jnp.bfloat16), vb1.reshape(1, -1),
      vw2.astype(jnp.bfloat16), vb2.reshape(1, -1),
      sw1.astype(jnp.bfloat16), sb1.reshape(1, -1),
      sw2.astype(jnp.bfloat16), sb2.reshape(1, -1))
    return out.reshape(B, Nv, 512)


# speedup vs baseline: 1.1892x; 1.0436x over previous
import jax
import jax.numpy as jnp
from jax import lax
from jax.experimental import pallas as pl
from jax.experimental.pallas import tpu as pltpu

_NSHELF = 64
_SPATIAL = 128
_SLOPE = 0.01
_T = 4096
_R = _T // 128
_C = 128


def _leaky(v):
    return jnp.maximum(v, _SLOPE * v)


def _body(st_ref, st2_ref, x_ref, vw1_ref, vb1_ref, vw2_ref, vb2_ref,
          sw1_ref, sb1_ref, sw2_ref, sb2_ref, out_ref):
    f32 = jnp.float32

    mf = (st2_ref[...] == _NSHELF).astype(f32)
    iu0 = lax.broadcasted_iota(jnp.int32, (128, 128), 0)
    iu1 = lax.broadcasted_iota(jnp.int32, (128, 128), 1)
    upper = jnp.where(iu0 <= iu1, 1.0, 0.0).astype(f32)
    rowcum = jnp.dot(mf, upper, preferred_element_type=f32)
    rowtot = rowcum[:, 127:128]
    is0 = lax.broadcasted_iota(jnp.int32, (_R, _R), 0)
    is1 = lax.broadcasted_iota(jnp.int32, (_R, _R), 1)
    strict = jnp.where(is1 < is0, 1.0, 0.0).astype(f32)
    prefix = jnp.dot(strict, rowtot, preferred_element_type=f32)
    rank = rowcum + prefix - mf
    rank = jnp.where(mf > 0, rank, -1.0)

    iota_c = lax.broadcasted_iota(jnp.int32, (_C, 128), 0).astype(f32)
    pieces = []
    for r in range(_R):
        rr = jnp.broadcast_to(rank[r:r + 1, :], (_C, 128))
        pieces.append(jnp.where(rr == iota_c, 1.0, 0.0))
    onehot = jnp.concatenate(pieces, axis=1).astype(f32)

    x = x_ref[...]
    gx = jnp.dot(onehot, x, preferred_element_type=f32)
    hv = _leaky(jnp.dot(gx[:, :_SPATIAL], vw1_ref[...],
                        preferred_element_type=f32) + vb1_ref[...])
    zv = jnp.dot(hv, vw2_ref[...], preferred_element_type=f32) + vb2_ref[...]

    hs = _leaky(jnp.dot(x, sw1_ref[...], preferred_element_type=f32)
                + sb1_ref[...])
    zs = jnp.dot(hs, sw2_ref[...], preferred_element_type=f32) + sb2_ref[...]

    iota_cl = lax.broadcasted_iota(jnp.int32, (128, _C), 1).astype(f32)
    rank_col = jnp.transpose(rank)
    pieces_t = []
    for r in range(_R):
        rr = jnp.broadcast_to(rank_col[:, r:r + 1], (128, _C))
        pieces_t.append(jnp.where(rr == iota_cl, 1.0, 0.0))
    onehot_t = jnp.concatenate(pieces_t, axis=0).astype(f32)
    scat = jnp.dot(onehot_t, zv, preferred_element_type=f32)
    mask_col = st_ref[...] == _NSHELF
    out_ref[...] = _leaky(jnp.where(mask_col, scat, zs))


def kernel(state, x, vw1, vb1, vw2, vb2, sw1, sb1, sw2, sb2):
    B, Nv, F = x.shape
    n_tok = B * Nv
    st = state.reshape(n_tok, 1).astype(jnp.int32)
    st2 = state.reshape(n_tok // 128, 128).astype(jnp.int32)
    xf = x.reshape(n_tok, F)
    grid = (n_tok // _T,)
    full = lambda shape: pl.BlockSpec(shape, lambda i: (0, 0))
    hbm = pl.BlockSpec(memory_space=pltpu.MemorySpace.HBM)
    out = pl.pallas_call(
        _body,
        grid=grid,
        in_specs=[
            pl.BlockSpec((_T, 1), lambda i: (i, 0)),
            pl.BlockSpec((_R, 128), lambda i: (i, 0)),
            pl.BlockSpec((_T, F), lambda i: (i, 0)),
            full(vw1.shape), full((1, vb1.shape[0])),
            full(vw2.shape), full((1, vb2.shape[0])),
            full(sw1.shape), full((1, sb1.shape[0])),
            full(sw2.shape), full((1, sb2.shape[0])),
        ],
        out_specs=pl.BlockSpec((_T, 512), lambda i: (i, 0)),
        out_shape=jax.ShapeDtypeStruct((n_tok, 512), jnp.float32),
        compiler_params=pltpu.CompilerParams(
            dimension_semantics=("parallel",)),
    )(st, st2, xf, vw1, vb1.reshape(1, -1), vw2, vb2.reshape(1, -1),
      sw1, sb1.reshape(1, -1), sw2, sb2.reshape(1, -1))
    return out.reshape(B, Nv, 512)
